# Initial kernel scaffold; baseline (speedup 1.0000x reference)
import functools

import jax
import jax.numpy as jnp
from jax import lax
from jax.experimental import pallas as pl
from jax.experimental.pallas import tpu as pltpu
from jax.experimental.pallas import tpu_sc as plsc

B, K_IN, K_OUT = 2048, 256, 256
IN_DIM, OUT_DIM = 2048, 8192

NC, NS, L = 2, 16, 16
NW = NC * NS
TOK_PER_W = B // NW

_MESH = plsc.VectorSubcoreMesh(core_axis_name="c", subcore_axis_name="s")


@functools.partial(
    pl.kernel,
    mesh=_MESH,
    out_type=jax.ShapeDtypeStruct((B, IN_DIM), jnp.float32),
    scratch_types=[
        pltpu.VMEM((K_IN,), jnp.int32),
        pltpu.VMEM((K_IN,), jnp.float32),
        pltpu.VMEM((IN_DIM,), jnp.float32),
    ],
)
def _scatter_kernel(vals_hbm, idx_hbm, x_hbm, idx_v, vals_v, row_v):
    wid = lax.axis_index("s") * NC + lax.axis_index("c")

    def token_body(t, carry):
        b = wid * TOK_PER_W + t
        pltpu.sync_copy(idx_hbm.at[b], idx_v)
        pltpu.sync_copy(vals_hbm.at[b], vals_v)

        def zero_body(i, c):
            row_v[pl.ds(i * L, L)] = jnp.zeros((L,), jnp.float32)
            return c

        lax.fori_loop(0, IN_DIM // L, zero_body, 0)

        def scat_body(g, c):
            iv = idx_v[pl.ds(g * L, L)]
            vv = vals_v[pl.ds(g * L, L)]
            plsc.addupdate_scatter(row_v, [iv], vv)
            return c

        lax.fori_loop(0, K_IN // L, scat_body, 0)
        pltpu.sync_copy(row_v, x_hbm.at[b])
        return carry

    lax.fori_loop(0, TOK_PER_W, token_body, 0)


_BM = 1024
_BN = 1024


def _mm_body(x_ref, w_ref, b_ref, y_ref):
    y_ref[...] = (
        lax.dot_general(
            x_ref[...],
            w_ref[...],
            dimension_numbers=(((1,), (1,)), ((), ())),
            preferred_element_type=jnp.float32,
        )
        + b_ref[...]
    )


def _matmul(x, w, bias2d):
    return pl.pallas_call(
        _mm_body,
        grid=(B // _BM, OUT_DIM // _BN),
        in_specs=[
            pl.BlockSpec((_BM, IN_DIM), lambda i, j: (i, 0)),
            pl.BlockSpec((_BN, IN_DIM), lambda i, j: (j, 0)),
            pl.BlockSpec((1, _BN), lambda i, j: (0, j)),
        ],
        out_specs=pl.BlockSpec((_BM, _BN), lambda i, j: (i, j)),
        out_shape=jax.ShapeDtypeStruct((B, OUT_DIM), jnp.float32),
    )(x, w, bias2d)


@functools.partial(
    pl.kernel,
    mesh=_MESH,
    out_type=jax.ShapeDtypeStruct((B, K_OUT), jnp.float32),
    scratch_types=[
        pltpu.VMEM((K_OUT,), jnp.int32),
        pltpu.VMEM((OUT_DIM,), jnp.float32),
        pltpu.VMEM((K_OUT,), jnp.float32),
    ],
)
def _gather_kernel(y_hbm, idx_hbm, out_hbm, idx_v, row_v, o_v):
    wid = lax.axis_index("s") * NC + lax.axis_index("c")

    def token_body(t, carry):
        b = wid * TOK_PER_W + t
        pltpu.sync_copy(idx_hbm.at[b], idx_v)
        pltpu.sync_copy(y_hbm.at[b], row_v)

        def gat_body(g, c):
            iv = idx_v[pl.ds(g * L, L)]
            o_v[pl.ds(g * L, L)] = plsc.load_gather(row_v, [iv])
            return c

        lax.fori_loop(0, K_OUT // L, gat_body, 0)
        pltpu.sync_copy(o_v, out_hbm.at[b])
        return carry

    lax.fori_loop(0, TOK_PER_W, token_body, 0)


def kernel(in_values, active_in_indices, active_out_indices, W, bias):
    in_values = in_values.astype(jnp.float32)
    idx_in = active_in_indices.astype(jnp.int32)
    idx_out = active_out_indices.astype(jnp.int32)
    W = W.astype(jnp.float32)
    bias2d = bias.astype(jnp.float32).reshape(1, OUT_DIM)

    x = _scatter_kernel(in_values, idx_in)
    y = _matmul(x, W, bias2d)
    out = _gather_kernel(y, idx_out)
    return out


# speedup vs baseline: 7785.1465x; 7785.1465x over previous
import functools

import jax
import jax.numpy as jnp
from jax import lax
from jax.experimental import pallas as pl
from jax.experimental.pallas import tpu as pltpu
from jax.experimental.pallas import tpu_sc as plsc

B, K_IN, K_OUT = 2048, 256, 256
IN_DIM, OUT_DIM = 2048, 8192

NC, NS, L = 2, 16, 16
NW = NC * NS
TOK_PER_W = B // NW

_MESH = plsc.VectorSubcoreMesh(core_axis_name="c", subcore_axis_name="s")
_SC_PARAMS = pltpu.CompilerParams(needs_layout_passes=False)


@functools.partial(
    pl.kernel,
    mesh=_MESH,
    out_type=jax.ShapeDtypeStruct((B, IN_DIM), jnp.float32),
    scratch_types=[
        pltpu.VMEM((K_IN,), jnp.int32),
        pltpu.VMEM((K_IN,), jnp.float32),
        pltpu.VMEM((IN_DIM,), jnp.float32),
    ],
    compiler_params=_SC_PARAMS,
)
def _scatter_kernel(vals_hbm, idx_hbm, x_hbm, idx_v, vals_v, row_v):
    wid = lax.axis_index("s") * NC + lax.axis_index("c")

    def token_body(t, carry):
        b = wid * TOK_PER_W + t
        pltpu.sync_copy(idx_hbm.at[b], idx_v)
        pltpu.sync_copy(vals_hbm.at[b], vals_v)

        def zero_body(i, c):
            row_v[pl.ds(i * L, L)] = jnp.zeros((L,), jnp.float32)
            return c

        lax.fori_loop(0, IN_DIM // L, zero_body, 0)

        def scat_body(g, c):
            iv = idx_v[pl.ds(g * L, L)]
            vv = vals_v[pl.ds(g * L, L)]
            plsc.addupdate_scatter(row_v, [iv], vv)
            return c

        lax.fori_loop(0, K_IN // L, scat_body, 0)
        pltpu.sync_copy(row_v, x_hbm.at[b])
        return carry

    lax.fori_loop(0, TOK_PER_W, token_body, 0)


_BM = 1024
_BN = 1024


def _mm_body(x_ref, w_ref, b_ref, y_ref):
    y_ref[...] = (
        lax.dot_general(
            x_ref[...],
            w_ref[...],
            dimension_numbers=(((1,), (1,)), ((), ())),
            preferred_element_type=jnp.float32,
        )
        + b_ref[...]
    )


def _matmul(x, w, bias2d):
    return pl.pallas_call(
        _mm_body,
        grid=(B // _BM, OUT_DIM // _BN),
        in_specs=[
            pl.BlockSpec((_BM, IN_DIM), lambda i, j: (i, 0)),
            pl.BlockSpec((_BN, IN_DIM), lambda i, j: (j, 0)),
            pl.BlockSpec((1, _BN), lambda i, j: (0, j)),
        ],
        out_specs=pl.BlockSpec((_BM, _BN), lambda i, j: (i, j)),
        out_shape=jax.ShapeDtypeStruct((B, OUT_DIM), jnp.float32),
    )(x, w, bias2d)


@functools.partial(
    pl.kernel,
    mesh=_MESH,
    out_type=jax.ShapeDtypeStruct((B, K_OUT), jnp.float32),
    scratch_types=[
        pltpu.VMEM((K_OUT,), jnp.int32),
        pltpu.VMEM((OUT_DIM,), jnp.float32),
        pltpu.VMEM((K_OUT,), jnp.float32),
    ],
    compiler_params=_SC_PARAMS,
)
def _gather_kernel(y_hbm, idx_hbm, out_hbm, idx_v, row_v, o_v):
    wid = lax.axis_index("s") * NC + lax.axis_index("c")

    def token_body(t, carry):
        b = wid * TOK_PER_W + t
        pltpu.sync_copy(idx_hbm.at[b], idx_v)
        pltpu.sync_copy(y_hbm.at[b], row_v)

        def gat_body(g, c):
            iv = idx_v[pl.ds(g * L, L)]
            o_v[pl.ds(g * L, L)] = plsc.load_gather(row_v, [iv])
            return c

        lax.fori_loop(0, K_OUT // L, gat_body, 0)
        pltpu.sync_copy(o_v, out_hbm.at[b])
        return carry

    lax.fori_loop(0, TOK_PER_W, token_body, 0)


def kernel(in_values, active_in_indices, active_out_indices, W, bias):
    in_values = in_values.astype(jnp.float32)
    idx_in = active_in_indices.astype(jnp.int32)
    idx_out = active_out_indices.astype(jnp.int32)
    W = W.astype(jnp.float32)
    bias2d = bias.astype(jnp.float32).reshape(1, OUT_DIM)

    x = _scatter_kernel(in_values, idx_in)
    y = _matmul(x, W, bias2d)
    out = _gather_kernel(y, idx_out)
    return out


# speedup vs baseline: 13153.5507x; 1.6896x over previous
import functools

import jax
import jax.numpy as jnp
from jax import lax
from jax.experimental import pallas as pl
from jax.experimental.pallas import tpu as pltpu
from jax.experimental.pallas import tpu_sc as plsc

B, K_IN, K_OUT = 2048, 256, 256
IN_DIM, OUT_DIM = 2048, 8192

NC, NS, L = 2, 16, 16
NW = NC * NS
TOK_PER_W = B // NW
CHUNK = 32

_MESH = plsc.VectorSubcoreMesh(core_axis_name="c", subcore_axis_name="s")
_SC_PARAMS = pltpu.CompilerParams(needs_layout_passes=False)

@functools.partial(
    pl.kernel,
    mesh=_MESH,
    out_type=jax.ShapeDtypeStruct((B, IN_DIM), jnp.float32),
    scratch_types=[
        pltpu.VMEM((TOK_PER_W, K_IN), jnp.int32),
        pltpu.VMEM((TOK_PER_W, K_IN), jnp.float32),
        pltpu.VMEM((CHUNK, IN_DIM), jnp.float32),
    ],
    compiler_params=_SC_PARAMS,
)
def _scatter_kernel(vals_hbm, idx_hbm, x_hbm, idx_v, vals_v, rows_v):
    wid = lax.axis_index("s") * NC + lax.axis_index("c")
    tok0 = wid * TOK_PER_W
    pltpu.sync_copy(idx_hbm.at[pl.ds(tok0, TOK_PER_W)], idx_v)
    pltpu.sync_copy(vals_hbm.at[pl.ds(tok0, TOK_PER_W)], vals_v)

    zeros = jnp.zeros((L,), jnp.float32)

    def zero_row(r, c):
        def zero_col(i, c2):
            rows_v[r, pl.ds(i * L, L)] = zeros
            return c2

        lax.fori_loop(0, IN_DIM // L, zero_col, 0)
        return c

    lax.fori_loop(0, CHUNK, zero_row, 0)

    def chunk_body(ci, c):
        def tok_scatter(tl, c2):
            t = ci * CHUNK + tl
            tvec = jnp.full((L,), tl, jnp.int32)
            for g in range(K_IN // L):
                iv = idx_v[t, pl.ds(g * L, L)]
                vv = vals_v[t, pl.ds(g * L, L)]
                plsc.addupdate_scatter(rows_v, [tvec, iv], vv)
            return c2

        lax.fori_loop(0, CHUNK, tok_scatter, 0)
        pltpu.sync_copy(rows_v, x_hbm.at[pl.ds(tok0 + ci * CHUNK, CHUNK)])

        def tok_rezero(tl, c2):
            t = ci * CHUNK + tl
            tvec = jnp.full((L,), tl, jnp.int32)
            for g in range(K_IN // L):
                iv = idx_v[t, pl.ds(g * L, L)]
                plsc.store_scatter(rows_v, [tvec, iv], zeros)
            return c2

        lax.fori_loop(0, CHUNK, tok_rezero, 0)
        return c

    lax.fori_loop(0, TOK_PER_W // CHUNK, chunk_body, 0)


_BM = 1024
_BN = 1024


def _mm_body(x_ref, w_ref, b_ref, y_ref):
    y_ref[...] = (
        lax.dot_general(
            x_ref[...],
            w_ref[...],
            dimension_numbers=(((1,), (1,)), ((), ())),
            preferred_element_type=jnp.float32,
        )
        + b_ref[...]
    )


def _matmul(x, w, bias2d):
    return pl.pallas_call(
        _mm_body,
        grid=(B // _BM, OUT_DIM // _BN),
        in_specs=[
            pl.BlockSpec((_BM, IN_DIM), lambda i, j: (i, 0)),
            pl.BlockSpec((_BN, IN_DIM), lambda i, j: (j, 0)),
            pl.BlockSpec((1, _BN), lambda i, j: (0, j)),
        ],
        out_specs=pl.BlockSpec((_BM, _BN), lambda i, j: (i, j)),
        out_shape=jax.ShapeDtypeStruct((B, OUT_DIM), jnp.float32),
    )(x, w, bias2d)


@functools.partial(
    pl.kernel,
    mesh=_MESH,
    out_type=jax.ShapeDtypeStruct((B, K_OUT), jnp.float32),
    scratch_types=[
        pltpu.VMEM((TOK_PER_W, K_OUT), jnp.int32),
        pltpu.VMEM((TOK_PER_W, K_OUT), jnp.float32),
        pltpu.VMEM((OUT_DIM,), jnp.float32),
        pltpu.VMEM((OUT_DIM,), jnp.float32),
        pltpu.SemaphoreType.DMA,
        pltpu.SemaphoreType.DMA,
    ],
    compiler_params=_SC_PARAMS,
)
def _gather_kernel(y_hbm, idx_hbm, out_hbm, idx_v, out_v, row0, row1, sem0, sem1):
    wid = lax.axis_index("s") * NC + lax.axis_index("c")
    tok0 = wid * TOK_PER_W
    pltpu.sync_copy(idx_hbm.at[pl.ds(tok0, TOK_PER_W)], idx_v)

    rows = (row0, row1)
    sems = (sem0, sem1)
    pltpu.async_copy(y_hbm.at[tok0], row0, sem0)
    pltpu.async_copy(y_hbm.at[tok0 + 1], row1, sem1)

    def pair_body(p, c):
        for bsel in range(2):
            t = p * 2 + bsel
            row, sem = rows[bsel], sems[bsel]
            pltpu.make_async_copy(y_hbm.at[tok0 + t], row, sem).wait()
            for g in range(K_OUT // L):
                iv = idx_v[t, pl.ds(g * L, L)]
                out_v[t, pl.ds(g * L, L)] = plsc.load_gather(row, [iv])

            @pl.when(t + 2 < TOK_PER_W)
            def _():
                pltpu.async_copy(y_hbm.at[tok0 + t + 2], row, sem)

        return c

    lax.fori_loop(0, TOK_PER_W // 2, pair_body, 0)
    pltpu.sync_copy(out_v, out_hbm.at[pl.ds(tok0, TOK_PER_W)])


def kernel(in_values, active_in_indices, active_out_indices, W, bias):
    in_values = in_values.astype(jnp.float32)
    idx_in = active_in_indices.astype(jnp.int32)
    idx_out = active_out_indices.astype(jnp.int32)
    W = W.astype(jnp.float32)
    bias2d = bias.astype(jnp.float32).reshape(1, OUT_DIM)

    x = _scatter_kernel(in_values, idx_in)
    y = _matmul(x, W, bias2d)
    out = _gather_kernel(y, idx_out)
    return out


# speedup vs baseline: 13321.4804x; 1.0128x over previous
import functools

import jax
import jax.numpy as jnp
from jax import lax
from jax.experimental import pallas as pl
from jax.experimental.pallas import tpu as pltpu
from jax.experimental.pallas import tpu_sc as plsc

B, K_IN, K_OUT = 2048, 256, 256
IN_DIM, OUT_DIM = 2048, 8192

NC, NS, L = 2, 16, 16
NW = NC * NS
TOK_PER_W = B // NW
CHUNK = 32

_MESH = plsc.VectorSubcoreMesh(core_axis_name="c", subcore_axis_name="s")
_SC_PARAMS = pltpu.CompilerParams(needs_layout_passes=False)

@functools.partial(
    pl.kernel,
    mesh=_MESH,
    out_type=jax.ShapeDtypeStruct((B, IN_DIM), jnp.float32),
    scratch_types=[
        pltpu.VMEM((TOK_PER_W, K_IN), jnp.int32),
        pltpu.VMEM((TOK_PER_W, K_IN), jnp.float32),
        pltpu.VMEM((CHUNK, IN_DIM), jnp.float32),
    ],
    compiler_params=_SC_PARAMS,
)
def _scatter_kernel(vals_hbm, idx_hbm, x_hbm, idx_v, vals_v, rows_v):
    wid = lax.axis_index("s") * NC + lax.axis_index("c")
    tok0 = wid * TOK_PER_W
    pltpu.sync_copy(idx_hbm.at[pl.ds(tok0, TOK_PER_W)], idx_v)
    pltpu.sync_copy(vals_hbm.at[pl.ds(tok0, TOK_PER_W)], vals_v)

    zeros = jnp.zeros((L,), jnp.float32)

    def zero_row(r, c):
        def zero_col(i, c2):
            rows_v[r, pl.ds(i * L, L)] = zeros
            return c2

        lax.fori_loop(0, IN_DIM // L, zero_col, 0)
        return c

    lax.fori_loop(0, CHUNK, zero_row, 0)

    def chunk_body(ci, c):
        def tok_scatter(tl, c2):
            t = ci * CHUNK + tl
            tvec = jnp.full((L,), tl, jnp.int32)
            for g in range(K_IN // L):
                iv = idx_v[t, pl.ds(g * L, L)]
                vv = vals_v[t, pl.ds(g * L, L)]
                plsc.addupdate_scatter(rows_v, [tvec, iv], vv)
            return c2

        lax.fori_loop(0, CHUNK, tok_scatter, 0)
        pltpu.sync_copy(rows_v, x_hbm.at[pl.ds(tok0 + ci * CHUNK, CHUNK)])

        def tok_rezero(tl, c2):
            t = ci * CHUNK + tl
            tvec = jnp.full((L,), tl, jnp.int32)
            for g in range(K_IN // L):
                iv = idx_v[t, pl.ds(g * L, L)]
                plsc.store_scatter(rows_v, [tvec, iv], zeros)
            return c2

        lax.fori_loop(0, CHUNK, tok_rezero, 0)
        return c

    lax.fori_loop(0, TOK_PER_W // CHUNK, chunk_body, 0)


_BM = B
_BN = 1024


def _mm_body(x_ref, w_ref, b_ref, y_ref):
    y_ref[...] = (
        lax.dot_general(
            x_ref[...],
            w_ref[...],
            dimension_numbers=(((1,), (1,)), ((), ())),
            preferred_element_type=jnp.float32,
        )
        + b_ref[...]
    )


def _matmul(x, w, bias2d):
    return pl.pallas_call(
        _mm_body,
        grid=(OUT_DIM // _BN,),
        in_specs=[
            pl.BlockSpec((_BM, IN_DIM), lambda j: (0, 0)),
            pl.BlockSpec((_BN, IN_DIM), lambda j: (j, 0)),
            pl.BlockSpec((1, _BN), lambda j: (0, j)),
        ],
        out_specs=pl.BlockSpec((_BM, _BN), lambda j: (0, j)),
        out_shape=jax.ShapeDtypeStruct((B, OUT_DIM), jnp.float32),
    )(x, w, bias2d)


@functools.partial(
    pl.kernel,
    mesh=_MESH,
    out_type=jax.ShapeDtypeStruct((B, K_OUT), jnp.float32),
    scratch_types=[
        pltpu.VMEM((TOK_PER_W, K_OUT), jnp.int32),
        pltpu.VMEM((TOK_PER_W, K_OUT), jnp.float32),
        pltpu.VMEM((OUT_DIM,), jnp.float32),
        pltpu.VMEM((OUT_DIM,), jnp.float32),
        pltpu.SemaphoreType.DMA,
        pltpu.SemaphoreType.DMA,
    ],
    compiler_params=_SC_PARAMS,
)
def _gather_kernel(y_hbm, idx_hbm, out_hbm, idx_v, out_v, row0, row1, sem0, sem1):
    wid = lax.axis_index("s") * NC + lax.axis_index("c")
    tok0 = wid * TOK_PER_W
    pltpu.sync_copy(idx_hbm.at[pl.ds(tok0, TOK_PER_W)], idx_v)

    rows = (row0, row1)
    sems = (sem0, sem1)
    pltpu.async_copy(y_hbm.at[tok0], row0, sem0)
    pltpu.async_copy(y_hbm.at[tok0 + 1], row1, sem1)

    def pair_body(p, c):
        for bsel in range(2):
            t = p * 2 + bsel
            row, sem = rows[bsel], sems[bsel]
            pltpu.make_async_copy(y_hbm.at[tok0 + t], row, sem).wait()
            for g in range(K_OUT // L):
                iv = idx_v[t, pl.ds(g * L, L)]
                out_v[t, pl.ds(g * L, L)] = plsc.load_gather(row, [iv])

            @pl.when(t + 2 < TOK_PER_W)
            def _():
                pltpu.async_copy(y_hbm.at[tok0 + t + 2], row, sem)

        return c

    lax.fori_loop(0, TOK_PER_W // 2, pair_body, 0)
    pltpu.sync_copy(out_v, out_hbm.at[pl.ds(tok0, TOK_PER_W)])


def kernel(in_values, active_in_indices, active_out_indices, W, bias):
    in_values = in_values.astype(jnp.float32)
    idx_in = active_in_indices.astype(jnp.int32)
    idx_out = active_out_indices.astype(jnp.int32)
    W = W.astype(jnp.float32)
    bias2d = bias.astype(jnp.float32).reshape(1, OUT_DIM)

    x = _scatter_kernel(in_values, idx_in)
    y = _matmul(x, W, bias2d)
    out = _gather_kernel(y, idx_out)
    return out
